# Initial kernel scaffold; baseline (speedup 1.0000x reference)
import jax
import jax.numpy as jnp
from jax.experimental import pallas as pl
from jax.experimental.pallas import tpu as pltpu

_NEG = -1e30


def _lrelu(x):
    return jnp.where(x >= 0, x, 0.2 * x)


def _softmax_rows(score, adj_bool):
    masked = jnp.where(adj_bool, score, _NEG)
    m = jnp.max(masked, axis=1, keepdims=True)
    e = jnp.exp(masked - m)
    return e / jnp.sum(e, axis=1, keepdims=True)


def _egat_kernel(idx_ref,
                 H_ref, ECL_ref, ME_ref, adjN_ref, adjE_ref,
                 WH_ref, WHb_ref, Wblk_ref, biasc_ref, SN_ref, WV_ref,
                 cst_ref, BE_ref, B3_ref,
                 Hn_ref, ECn_ref, ELn_ref, Hm_ref,
                 attnC_sc, attnL_sc):
    N = 64
    M = 50
    D = 256

    H = H_ref[0]
    ECL = ECL_ref[0]
    biasc = biasc_ref[...]

    Ht = jnp.dot(H, WH_ref[...], preferred_element_type=jnp.float32) + WHb_ref[...]
    FCL = jnp.dot(ECL, Wblk_ref[...], preferred_element_type=jnp.float32)
    fcfl = jnp.dot(ECL, WV_ref[...], preferred_element_type=jnp.float32)

    ME2 = ME_ref[0].reshape(N * N, M)
    w2 = jnp.dot(ME2, fcfl[:M, :], preferred_element_type=jnp.float32)
    w3 = w2.reshape(N, N, 2)
    s = jnp.dot(Ht, SN_ref[...], preferred_element_type=jnp.float32)
    cst = cst_ref[...]
    adjN = adjN_ref[...] > 0
    scC = _lrelu(s[:, 0:1] + jnp.broadcast_to(s[:, 1], (N, N)) + w3[:, :, 0] + cst[0:1, 0:1])
    scL = _lrelu(s[:, 2:3] + jnp.broadcast_to(s[:, 3], (N, N)) + w3[:, :, 1] + cst[0:1, 1:2])
    attnC = _softmax_rows(scC, adjN)
    attnL = _softmax_rows(scL, adjN)
    attnC_sc[...] = attnC
    attnL_sc[...] = attnL

    aggC = jnp.dot(attnC, Ht, preferred_element_type=jnp.float32)
    aggL = jnp.dot(attnL, Ht, preferred_element_type=jnp.float32)

    iota = jax.lax.broadcasted_iota(jnp.int32, (N, 1), 0)
    pm = jnp.zeros((N, 1), dtype=jnp.bool_)
    for k in range(8):
        pm = pm | (iota == idx_ref[k])
    any_adj = jnp.max(adjN_ref[...], axis=1, keepdims=True) > 0
    rm = pm & any_adj
    Hn_ref[0] = jnp.where(rm, 0.5 * (aggC + aggL), Ht)

    Hm_ref[0] = jnp.zeros((N, D), dtype=jnp.float32)
    FCL50 = FCL[:M, :]
    for k in range(8):
        i = idx_ref[k]
        rME = ME_ref[0, i]
        ECLr = jnp.dot(rME, FCL50, preferred_element_type=jnp.float32)
        XC = (ECLr[:, :D] + biasc[:, :D]) * Ht
        XL = (ECLr[:, D:] + biasc[:, D:]) * Ht
        aCr = attnC_sc[pl.ds(i, 1), :]
        aLr = attnL_sc[pl.ds(i, 1), :]
        mk = 0.5 * (jnp.dot(aCr, XC, preferred_element_type=jnp.float32)
                    + jnp.dot(aLr, XL, preferred_element_type=jnp.float32))
        Hm_ref[0, pl.ds(i, 1), :] = mk
    Hm_ref[0] = Hm_ref[0] * rm.astype(jnp.float32)

    ECLt = FCL + biasc
    xy = jnp.dot(ECLt, BE_ref[...], preferred_element_type=jnp.float32)
    zz = jnp.dot(H[0:1, :], B3_ref[...], preferred_element_type=jnp.float32)
    adjE = adjE_ref[...] > 0
    sEC = _lrelu(xy[:, 0:1] + jnp.broadcast_to(xy[:, 1], (N, N)) + zz[0:1, 0:1])
    sEL = _lrelu(xy[:, 2:3] + jnp.broadcast_to(xy[:, 3], (N, N)) + zz[0:1, 1:2])
    attnEC = _softmax_rows(sEC, adjE)
    attnEL = _softmax_rows(sEL, adjE)
    ECt = ECLt[:, :D]
    ELt = ECLt[:, D:]
    aggEC = jnp.dot(attnEC, ECt, preferred_element_type=jnp.float32)
    aggEL = jnp.dot(attnEL, ELt, preferred_element_type=jnp.float32)

    pmE = jnp.zeros((N, 1), dtype=jnp.bool_)
    for k in range(8):
        pmE = pmE | (iota == idx_ref[8 + k])
    any_adjE = jnp.max(adjE_ref[...], axis=1, keepdims=True) > 0
    rmE = pmE & any_adjE
    ECn_ref[0] = jnp.where(rmE, aggEC, ECt)
    ELn_ref[0] = jnp.where(rmE, aggEL, ELt)


def kernel(H, E_C, E_L, AH, AE, ME, MH, path_node_indices, path_edge_indices,
           W_H_w, W_H_b, W_EC_w, W_EC_b, W_EL_w, W_EL_b, a_C, a_L, b_C, b_L):
    B, N, ND = H.shape
    M = E_C.shape[1]
    Fe = E_C.shape[2]
    D = W_H_w.shape[1]
    f32 = jnp.float32

    a1C, a2C, a3C = a_C[0:D, 0], a_C[D:2 * D, 0], a_C[2 * D:, 0]
    a1L, a2L, a3L = a_L[0:D, 0], a_L[D:2 * D, 0], a_L[2 * D:, 0]
    b1C, b2C, b3C = b_C[0:D, 0], b_C[D:2 * D, 0], b_C[2 * D:, 0]
    b1L, b2L, b3L = b_L[0:D, 0], b_L[D:2 * D, 0], b_L[2 * D:, 0]

    SN = jnp.stack([a1C, a2C, a1L, a2L], axis=1)
    wvC = W_EC_w @ a3C
    wvL = W_EL_w @ a3L
    WV = jnp.zeros((2 * Fe, 2), f32).at[:Fe, 0].set(wvC).at[Fe:, 1].set(wvL)
    cst = jnp.stack([W_EC_b @ a3C, W_EL_b @ a3L]).reshape(1, 2)
    Wblk = jnp.zeros((2 * Fe, 2 * D), f32).at[:Fe, :D].set(W_EC_w).at[Fe:, D:].set(W_EL_w)
    biasc = jnp.concatenate([W_EC_b, W_EL_b]).reshape(1, 2 * D)
    BE = jnp.zeros((2 * D, 4), f32).at[:D, 0].set(b1C).at[:D, 1].set(b2C) \
                                   .at[D:, 2].set(b1L).at[D:, 3].set(b2L)
    B3 = jnp.stack([b3C, b3L], axis=1)
    WHb = W_H_b.reshape(1, D)

    pad_rows = N - M
    E_Cp = jnp.pad(E_C, ((0, 0), (0, pad_rows), (0, 0)))
    E_Lp = jnp.pad(E_L, ((0, 0), (0, pad_rows), (0, 0)))
    ECL = jnp.concatenate([E_Cp, E_Lp], axis=2)
    ME3 = ME.reshape(B, N, N, M)
    adjN = AH[0].astype(jnp.int32)
    adjE = jnp.pad(AE[0].astype(jnp.int32), ((0, pad_rows), (0, pad_rows)))
    idxs = jnp.concatenate([path_node_indices.astype(jnp.int32),
                            path_edge_indices.astype(jnp.int32)])

    grid_spec = pltpu.PrefetchScalarGridSpec(
        num_scalar_prefetch=1,
        grid=(B,),
        in_specs=[
            pl.BlockSpec((1, N, ND), lambda b, idx: (b, 0, 0)),
            pl.BlockSpec((1, N, 2 * Fe), lambda b, idx: (b, 0, 0)),
            pl.BlockSpec((1, N, N, M), lambda b, idx: (b, 0, 0, 0)),
            pl.BlockSpec((N, N), lambda b, idx: (0, 0)),
            pl.BlockSpec((N, N), lambda b, idx: (0, 0)),
            pl.BlockSpec((ND, D), lambda b, idx: (0, 0)),
            pl.BlockSpec((1, D), lambda b, idx: (0, 0)),
            pl.BlockSpec((2 * Fe, 2 * D), lambda b, idx: (0, 0)),
            pl.BlockSpec((1, 2 * D), lambda b, idx: (0, 0)),
            pl.BlockSpec((D, 4), lambda b, idx: (0, 0)),
            pl.BlockSpec((2 * Fe, 2), lambda b, idx: (0, 0)),
            pl.BlockSpec((1, 2), lambda b, idx: (0, 0)),
            pl.BlockSpec((2 * D, 4), lambda b, idx: (0, 0)),
            pl.BlockSpec((D, 2), lambda b, idx: (0, 0)),
        ],
        out_specs=[
            pl.BlockSpec((1, N, D), lambda b, idx: (b, 0, 0)),
            pl.BlockSpec((1, N, D), lambda b, idx: (b, 0, 0)),
            pl.BlockSpec((1, N, D), lambda b, idx: (b, 0, 0)),
            pl.BlockSpec((1, N, D), lambda b, idx: (b, 0, 0)),
        ],
        scratch_shapes=[
            pltpu.VMEM((N, N), f32),
            pltpu.VMEM((N, N), f32),
        ],
    )

    out_shape = [
        jax.ShapeDtypeStruct((B, N, D), f32),
        jax.ShapeDtypeStruct((B, N, D), f32),
        jax.ShapeDtypeStruct((B, N, D), f32),
        jax.ShapeDtypeStruct((B, N, D), f32),
    ]

    Hn, ECn, ELn, Hm = pl.pallas_call(
        _egat_kernel,
        grid_spec=grid_spec,
        out_shape=out_shape,
        compiler_params=pltpu.CompilerParams(
            dimension_semantics=("arbitrary",),
        ),
    )(idxs, H, ECL, ME3, adjN, adjE, W_H_w, WHb, Wblk, biasc, SN, WV, cst, BE, B3)

    return (Hn, ECn[:, :M, :], ELn[:, :M, :], Hm)


# speedup vs baseline: 5.2229x; 5.2229x over previous
import jax
import jax.numpy as jnp
from jax.experimental import pallas as pl
from jax.experimental.pallas import tpu as pltpu

_NEG = -1e30


def _lrelu(x):
    return jnp.where(x >= 0, x, 0.2 * x)


def _softmax_rows(score, adj_bool):
    masked = jnp.where(adj_bool, score, _NEG)
    m = jnp.max(masked, axis=1, keepdims=True)
    e = jnp.exp(masked - m)
    return e / jnp.sum(e, axis=1, keepdims=True)


def _egat_kernel(idx_ref,
                 H_ref, ECL_ref, ME_ref, adjN_ref, adjE_ref,
                 WH_ref, WHb_ref, Wblk_ref, biasc_ref, SN_ref, WV_ref,
                 cst_ref, BE_ref, B3_ref,
                 Hn_ref, ECn_ref, ELn_ref, Hm_ref,
                 attnC_sc, attnL_sc):
    N = 64
    M = 50
    D = 256

    H = H_ref[0]
    ECL = ECL_ref[0]
    biasc = biasc_ref[...]

    Ht = jnp.dot(H, WH_ref[...], preferred_element_type=jnp.float32) + WHb_ref[...]
    FCL = jnp.dot(ECL, Wblk_ref[...], preferred_element_type=jnp.float32)
    fcfl = jnp.dot(ECL, WV_ref[...], preferred_element_type=jnp.float32)

    ME2 = ME_ref[0].reshape(N * N, M)
    w2 = jnp.dot(ME2, fcfl[:M, :], preferred_element_type=jnp.float32)
    w3 = w2.reshape(N, N, 2)
    s = jnp.dot(Ht, SN_ref[...], preferred_element_type=jnp.float32)
    cst = cst_ref[...]
    adjN = adjN_ref[...] > 0
    scC = _lrelu(s[:, 0:1] + jnp.broadcast_to(s[:, 1], (N, N)) + w3[:, :, 0] + cst[0:1, 0:1])
    scL = _lrelu(s[:, 2:3] + jnp.broadcast_to(s[:, 3], (N, N)) + w3[:, :, 1] + cst[0:1, 1:2])
    attnC = _softmax_rows(scC, adjN)
    attnL = _softmax_rows(scL, adjN)
    attnC_sc[...] = attnC
    attnL_sc[...] = attnL

    aggC = jnp.dot(attnC, Ht, preferred_element_type=jnp.float32)
    aggL = jnp.dot(attnL, Ht, preferred_element_type=jnp.float32)

    iota = jax.lax.broadcasted_iota(jnp.int32, (N, 1), 0)
    pm = jnp.zeros((N, 1), dtype=jnp.bool_)
    for k in range(8):
        pm = pm | (iota == idx_ref[k])
    any_adj = jnp.max(adjN_ref[...], axis=1, keepdims=True) > 0
    rm = pm & any_adj
    Hn_ref[0] = jnp.where(rm, 0.5 * (aggC + aggL), Ht)

    Hm_ref[0] = jnp.zeros((N, D), dtype=jnp.float32)
    FCL50 = FCL[:M, :]
    for k in range(8):
        i = idx_ref[k]
        rME = ME_ref[0, i]
        ECLr = jnp.dot(rME, FCL50, preferred_element_type=jnp.float32)
        XC = (ECLr[:, :D] + biasc[:, :D]) * Ht
        XL = (ECLr[:, D:] + biasc[:, D:]) * Ht
        aCr = attnC_sc[pl.ds(i, 1), :]
        aLr = attnL_sc[pl.ds(i, 1), :]
        mk = 0.5 * (jnp.dot(aCr, XC, preferred_element_type=jnp.float32)
                    + jnp.dot(aLr, XL, preferred_element_type=jnp.float32))
        Hm_ref[0, pl.ds(i, 1), :] = mk
    Hm_ref[0] = Hm_ref[0] * rm.astype(jnp.float32)

    ECLt = FCL + biasc
    xy = jnp.dot(ECLt, BE_ref[...], preferred_element_type=jnp.float32)
    zz = jnp.dot(H[0:1, :], B3_ref[...], preferred_element_type=jnp.float32)
    adjE = adjE_ref[...] > 0
    sEC = _lrelu(xy[:, 0:1] + jnp.broadcast_to(xy[:, 1], (N, N)) + zz[0:1, 0:1])
    sEL = _lrelu(xy[:, 2:3] + jnp.broadcast_to(xy[:, 3], (N, N)) + zz[0:1, 1:2])
    attnEC = _softmax_rows(sEC, adjE)
    attnEL = _softmax_rows(sEL, adjE)
    ECt = ECLt[:, :D]
    ELt = ECLt[:, D:]
    aggEC = jnp.dot(attnEC, ECt, preferred_element_type=jnp.float32)
    aggEL = jnp.dot(attnEL, ELt, preferred_element_type=jnp.float32)

    pmE = jnp.zeros((N, 1), dtype=jnp.bool_)
    for k in range(8):
        pmE = pmE | (iota == idx_ref[8 + k])
    any_adjE = jnp.max(adjE_ref[...], axis=1, keepdims=True) > 0
    rmE = pmE & any_adjE
    ECn_ref[0] = jnp.where(rmE, aggEC, ECt)
    ELn_ref[0] = jnp.where(rmE, aggEL, ELt)


def kernel(H, E_C, E_L, AH, AE, ME, MH, path_node_indices, path_edge_indices,
           W_H_w, W_H_b, W_EC_w, W_EC_b, W_EL_w, W_EL_b, a_C, a_L, b_C, b_L):
    B, N, ND = H.shape
    M = E_C.shape[1]
    Fe = E_C.shape[2]
    D = W_H_w.shape[1]
    f32 = jnp.float32

    a1C, a2C, a3C = a_C[0:D, 0], a_C[D:2 * D, 0], a_C[2 * D:, 0]
    a1L, a2L, a3L = a_L[0:D, 0], a_L[D:2 * D, 0], a_L[2 * D:, 0]
    b1C, b2C, b3C = b_C[0:D, 0], b_C[D:2 * D, 0], b_C[2 * D:, 0]
    b1L, b2L, b3L = b_L[0:D, 0], b_L[D:2 * D, 0], b_L[2 * D:, 0]

    SN = jnp.stack([a1C, a2C, a1L, a2L], axis=1)
    hp = jax.lax.Precision.HIGHEST
    wvC = jnp.dot(W_EC_w, a3C, precision=None)
    wvL = jnp.dot(W_EL_w, a3L, precision=None)
    WV = jnp.zeros((2 * Fe, 2), f32).at[:Fe, 0].set(wvC).at[Fe:, 1].set(wvL)
    cst = jnp.stack([jnp.dot(W_EC_b, a3C, precision=None), jnp.dot(W_EL_b, a3L, precision=None)]).reshape(1, 2)
    Wblk = jnp.zeros((2 * Fe, 2 * D), f32).at[:Fe, :D].set(W_EC_w).at[Fe:, D:].set(W_EL_w)
    biasc = jnp.concatenate([W_EC_b, W_EL_b]).reshape(1, 2 * D)
    BE = jnp.zeros((2 * D, 4), f32).at[:D, 0].set(b1C).at[:D, 1].set(b2C) \
                                   .at[D:, 2].set(b1L).at[D:, 3].set(b2L)
    B3 = jnp.stack([b3C, b3L], axis=1)
    WHb = W_H_b.reshape(1, D)

    pad_rows = N - M
    E_Cp = jnp.pad(E_C, ((0, 0), (0, pad_rows), (0, 0)))
    E_Lp = jnp.pad(E_L, ((0, 0), (0, pad_rows), (0, 0)))
    ECL = jnp.concatenate([E_Cp, E_Lp], axis=2)
    ME3 = ME.reshape(B, N, N, M)
    adjN = AH[0].astype(jnp.int32)
    adjE = jnp.pad(AE[0].astype(jnp.int32), ((0, pad_rows), (0, pad_rows)))
    idxs = jnp.concatenate([path_node_indices.astype(jnp.int32),
                            path_edge_indices.astype(jnp.int32)])

    grid_spec = pltpu.PrefetchScalarGridSpec(
        num_scalar_prefetch=1,
        grid=(B,),
        in_specs=[
            pl.BlockSpec((1, N, ND), lambda b, idx: (b, 0, 0)),
            pl.BlockSpec((1, N, 2 * Fe), lambda b, idx: (b, 0, 0)),
            pl.BlockSpec((1, N, N, M), lambda b, idx: (b, 0, 0, 0)),
            pl.BlockSpec((N, N), lambda b, idx: (0, 0)),
            pl.BlockSpec((N, N), lambda b, idx: (0, 0)),
            pl.BlockSpec((ND, D), lambda b, idx: (0, 0)),
            pl.BlockSpec((1, D), lambda b, idx: (0, 0)),
            pl.BlockSpec((2 * Fe, 2 * D), lambda b, idx: (0, 0)),
            pl.BlockSpec((1, 2 * D), lambda b, idx: (0, 0)),
            pl.BlockSpec((D, 4), lambda b, idx: (0, 0)),
            pl.BlockSpec((2 * Fe, 2), lambda b, idx: (0, 0)),
            pl.BlockSpec((1, 2), lambda b, idx: (0, 0)),
            pl.BlockSpec((2 * D, 4), lambda b, idx: (0, 0)),
            pl.BlockSpec((D, 2), lambda b, idx: (0, 0)),
        ],
        out_specs=[
            pl.BlockSpec((1, N, D), lambda b, idx: (b, 0, 0)),
            pl.BlockSpec((1, N, D), lambda b, idx: (b, 0, 0)),
            pl.BlockSpec((1, N, D), lambda b, idx: (b, 0, 0)),
            pl.BlockSpec((1, N, D), lambda b, idx: (b, 0, 0)),
        ],
        scratch_shapes=[
            pltpu.VMEM((N, N), f32),
            pltpu.VMEM((N, N), f32),
        ],
    )

    out_shape = [
        jax.ShapeDtypeStruct((B, N, D), f32),
        jax.ShapeDtypeStruct((B, N, D), f32),
        jax.ShapeDtypeStruct((B, N, D), f32),
        jax.ShapeDtypeStruct((B, N, D), f32),
    ]

    Hn, ECn, ELn, Hm = pl.pallas_call(
        _egat_kernel,
        grid_spec=grid_spec,
        out_shape=out_shape,
        compiler_params=pltpu.CompilerParams(
            dimension_semantics=("arbitrary",),
        ),
    )(idxs, H, ECL, ME3, adjN, adjE, W_H_w, WHb, Wblk, biasc, SN, WV, cst, BE, B3)

    return (Hn, ECn[:, :M, :], ELn[:, :M, :], Hm)


# speedup vs baseline: 5.2313x; 1.0016x over previous
import jax
import jax.numpy as jnp
from jax.experimental import pallas as pl
from jax.experimental.pallas import tpu as pltpu

_NEG = -1e30


def _lrelu(x):
    return jnp.where(x >= 0, x, 0.2 * x)


def _softmax_rows(score, adj_bool):
    masked = jnp.where(adj_bool, score, _NEG)
    m = jnp.max(masked, axis=1, keepdims=True)
    e = jnp.exp(masked - m)
    return e / jnp.sum(e, axis=1, keepdims=True)


def _egat_kernel(idx_ref,
                 H_ref, ECL_ref, ME_ref, adjN_ref, adjE_ref,
                 WH_ref, WHb_ref, Wblk_ref, biasc_ref, SN_ref, WV_ref,
                 cst_ref, BE_ref, B3_ref,
                 Hn_ref, ECn_ref, ELn_ref, Hm_ref,
                 attnC_sc, attnL_sc):
    N = 64
    M = 50
    D = 256

    H = H_ref[0]
    ECL = ECL_ref[0]
    biasc = biasc_ref[...]

    Ht = jnp.dot(H, WH_ref[...], preferred_element_type=jnp.float32) + WHb_ref[...]
    FCL = jnp.dot(ECL, Wblk_ref[...], preferred_element_type=jnp.float32)
    fcfl = jnp.dot(ECL, WV_ref[...], preferred_element_type=jnp.float32)

    ME2 = ME_ref[0].reshape(N * N, M)
    w2 = jnp.dot(ME2, fcfl[:M, :], preferred_element_type=jnp.float32)
    w3 = w2.reshape(N, N, 2)
    s = jnp.dot(Ht, SN_ref[...], preferred_element_type=jnp.float32)
    cst = cst_ref[...]
    adjN = adjN_ref[...] > 0
    scC = _lrelu(s[:, 0:1] + jnp.broadcast_to(s[:, 1], (N, N)) + w3[:, :, 0] + cst[0:1, 0:1])
    scL = _lrelu(s[:, 2:3] + jnp.broadcast_to(s[:, 3], (N, N)) + w3[:, :, 1] + cst[0:1, 1:2])
    attnC = _softmax_rows(scC, adjN)
    attnL = _softmax_rows(scL, adjN)
    attnC_sc[...] = attnC
    attnL_sc[...] = attnL

    aggC = jnp.dot(attnC, Ht, preferred_element_type=jnp.float32)
    aggL = jnp.dot(attnL, Ht, preferred_element_type=jnp.float32)

    iota = jax.lax.broadcasted_iota(jnp.int32, (N, 1), 0)
    pm = jnp.zeros((N, 1), dtype=jnp.bool_)
    for k in range(8):
        pm = pm | (iota == idx_ref[k])
    any_adj = jnp.max(adjN_ref[...], axis=1, keepdims=True) > 0
    rm = pm & any_adj
    Hn_ref[0] = jnp.where(rm, 0.5 * (aggC + aggL), Ht)

    Hm_ref[0] = jnp.zeros((N, D), dtype=jnp.float32)
    FCL50 = FCL[:M, :]
    for k in range(8):
        i = idx_ref[k]
        rME = ME_ref[0, i]
        ECLr = jnp.dot(rME, FCL50, preferred_element_type=jnp.float32)
        XC = (ECLr[:, :D] + biasc[:, :D]) * Ht
        XL = (ECLr[:, D:] + biasc[:, D:]) * Ht
        aCr = attnC_sc[pl.ds(i, 1), :]
        aLr = attnL_sc[pl.ds(i, 1), :]
        mk = 0.5 * (jnp.dot(aCr, XC, preferred_element_type=jnp.float32)
                    + jnp.dot(aLr, XL, preferred_element_type=jnp.float32))
        Hm_ref[0, pl.ds(i, 1), :] = mk
    Hm_ref[0] = Hm_ref[0] * rm.astype(jnp.float32)

    ECLt = FCL + biasc
    xy = jnp.dot(ECLt, BE_ref[...], preferred_element_type=jnp.float32)
    zz = jnp.dot(H[0:1, :], B3_ref[...], preferred_element_type=jnp.float32)
    adjE = adjE_ref[...] > 0
    sEC = _lrelu(xy[:, 0:1] + jnp.broadcast_to(xy[:, 1], (N, N)) + zz[0:1, 0:1])
    sEL = _lrelu(xy[:, 2:3] + jnp.broadcast_to(xy[:, 3], (N, N)) + zz[0:1, 1:2])
    attnEC = _softmax_rows(sEC, adjE)
    attnEL = _softmax_rows(sEL, adjE)
    ECt = ECLt[:, :D]
    ELt = ECLt[:, D:]
    aggEC = jnp.dot(attnEC, ECt, preferred_element_type=jnp.float32)
    aggEL = jnp.dot(attnEL, ELt, preferred_element_type=jnp.float32)

    pmE = jnp.zeros((N, 1), dtype=jnp.bool_)
    for k in range(8):
        pmE = pmE | (iota == idx_ref[8 + k])
    any_adjE = jnp.max(adjE_ref[...], axis=1, keepdims=True) > 0
    rmE = pmE & any_adjE
    ECn_ref[0] = jnp.where(rmE, aggEC, ECt)
    ELn_ref[0] = jnp.where(rmE, aggEL, ELt)


def kernel(H, E_C, E_L, AH, AE, ME, MH, path_node_indices, path_edge_indices,
           W_H_w, W_H_b, W_EC_w, W_EC_b, W_EL_w, W_EL_b, a_C, a_L, b_C, b_L):
    B, N, ND = H.shape
    M = E_C.shape[1]
    Fe = E_C.shape[2]
    D = W_H_w.shape[1]
    f32 = jnp.float32

    a1C, a2C, a3C = a_C[0:D, 0], a_C[D:2 * D, 0], a_C[2 * D:, 0]
    a1L, a2L, a3L = a_L[0:D, 0], a_L[D:2 * D, 0], a_L[2 * D:, 0]
    b1C, b2C, b3C = b_C[0:D, 0], b_C[D:2 * D, 0], b_C[2 * D:, 0]
    b1L, b2L, b3L = b_L[0:D, 0], b_L[D:2 * D, 0], b_L[2 * D:, 0]

    SN = jnp.stack([a1C, a2C, a1L, a2L], axis=1)
    hp = jax.lax.Precision.HIGHEST
    wvC = jnp.dot(W_EC_w, a3C, precision=None)
    wvL = jnp.dot(W_EL_w, a3L, precision=None)
    WV = jnp.zeros((2 * Fe, 2), f32).at[:Fe, 0].set(wvC).at[Fe:, 1].set(wvL)
    cst = jnp.stack([jnp.dot(W_EC_b, a3C, precision=None), jnp.dot(W_EL_b, a3L, precision=None)]).reshape(1, 2)
    Wblk = jnp.zeros((2 * Fe, 2 * D), f32).at[:Fe, :D].set(W_EC_w).at[Fe:, D:].set(W_EL_w)
    biasc = jnp.concatenate([W_EC_b, W_EL_b]).reshape(1, 2 * D)
    BE = jnp.zeros((2 * D, 4), f32).at[:D, 0].set(b1C).at[:D, 1].set(b2C) \
                                   .at[D:, 2].set(b1L).at[D:, 3].set(b2L)
    B3 = jnp.stack([b3C, b3L], axis=1)
    WHb = W_H_b.reshape(1, D)

    pad_rows = N - M
    E_Cp = jnp.pad(E_C, ((0, 0), (0, pad_rows), (0, 0)))
    E_Lp = jnp.pad(E_L, ((0, 0), (0, pad_rows), (0, 0)))
    ECL = jnp.concatenate([E_Cp, E_Lp], axis=2)
    ME3 = ME.reshape(B, N, N, M)
    adjN = AH[0].astype(jnp.int32)
    adjE = jnp.pad(AE[0].astype(jnp.int32), ((0, pad_rows), (0, pad_rows)))
    idxs = jnp.concatenate([path_node_indices.astype(jnp.int32),
                            path_edge_indices.astype(jnp.int32)])

    grid_spec = pltpu.PrefetchScalarGridSpec(
        num_scalar_prefetch=1,
        grid=(B,),
        in_specs=[
            pl.BlockSpec((1, N, ND), lambda b, idx: (b, 0, 0)),
            pl.BlockSpec((1, N, 2 * Fe), lambda b, idx: (b, 0, 0)),
            pl.BlockSpec((1, N, N, M), lambda b, idx: (b, 0, 0, 0)),
            pl.BlockSpec((N, N), lambda b, idx: (0, 0)),
            pl.BlockSpec((N, N), lambda b, idx: (0, 0)),
            pl.BlockSpec((ND, D), lambda b, idx: (0, 0)),
            pl.BlockSpec((1, D), lambda b, idx: (0, 0)),
            pl.BlockSpec((2 * Fe, 2 * D), lambda b, idx: (0, 0)),
            pl.BlockSpec((1, 2 * D), lambda b, idx: (0, 0)),
            pl.BlockSpec((D, 4), lambda b, idx: (0, 0)),
            pl.BlockSpec((2 * Fe, 2), lambda b, idx: (0, 0)),
            pl.BlockSpec((1, 2), lambda b, idx: (0, 0)),
            pl.BlockSpec((2 * D, 4), lambda b, idx: (0, 0)),
            pl.BlockSpec((D, 2), lambda b, idx: (0, 0)),
        ],
        out_specs=[
            pl.BlockSpec((1, N, D), lambda b, idx: (b, 0, 0)),
            pl.BlockSpec((1, N, D), lambda b, idx: (b, 0, 0)),
            pl.BlockSpec((1, N, D), lambda b, idx: (b, 0, 0)),
            pl.BlockSpec((1, N, D), lambda b, idx: (b, 0, 0)),
        ],
        scratch_shapes=[
            pltpu.VMEM((N, N), f32),
            pltpu.VMEM((N, N), f32),
        ],
    )

    out_shape = [
        jax.ShapeDtypeStruct((B, N, D), f32),
        jax.ShapeDtypeStruct((B, N, D), f32),
        jax.ShapeDtypeStruct((B, N, D), f32),
        jax.ShapeDtypeStruct((B, N, D), f32),
    ]

    Hn, ECn, ELn, Hm = pl.pallas_call(
        _egat_kernel,
        grid_spec=grid_spec,
        out_shape=out_shape,
        compiler_params=pltpu.CompilerParams(
            dimension_semantics=("parallel",),
        ),
    )(idxs, H, ECL, ME3, adjN, adjE, W_H_w, WHb, Wblk, biasc, SN, WV, cst, BE, B3)

    return (Hn, ECn[:, :M, :], ELn[:, :M, :], Hm)


# speedup vs baseline: 5.8154x; 1.1116x over previous
import jax
import jax.numpy as jnp
from jax.experimental import pallas as pl
from jax.experimental.pallas import tpu as pltpu

_NEG = -1e30


def _lrelu(x):
    return jnp.where(x >= 0, x, 0.2 * x)


def _softmax_rows(score, adj_bool):
    masked = jnp.where(adj_bool, score, _NEG)
    m = jnp.max(masked, axis=1, keepdims=True)
    e = jnp.exp(masked - m)
    return e / jnp.sum(e, axis=1, keepdims=True)


def _egat_kernel(pni_ref, pei_ref,
                 H_ref, EC_ref, EL_ref, AH_ref, AE_ref, ME_ref,
                 WH_ref, WHb_ref, WEC_ref, WECb_ref, WEL_ref, WELb_ref,
                 aC_ref, aL_ref, bC_ref, bL_ref,
                 Hn_ref, ECn_ref, ELn_ref, Hm_ref,
                 attnC_sc, attnL_sc):
    N = 64
    M = 50
    D = 256

    H = H_ref[0]
    EC = EC_ref[0]
    EL = EL_ref[0]
    ME2 = ME_ref[0]
    aC = aC_ref[...]
    aL = aL_ref[...]
    bC = bC_ref[...]
    bL = bL_ref[...]
    WECb = WECb_ref[...].reshape(1, D)
    WELb = WELb_ref[...].reshape(1, D)

    Ht = jnp.dot(H, WH_ref[...], preferred_element_type=jnp.float32) + WHb_ref[...].reshape(1, D)
    FC = jnp.dot(EC, WEC_ref[...], preferred_element_type=jnp.float32)
    FL = jnp.dot(EL, WEL_ref[...], preferred_element_type=jnp.float32)

    wvC = jnp.dot(WEC_ref[...], aC[2 * D:, :], preferred_element_type=jnp.float32)
    wvL = jnp.dot(WEL_ref[...], aL[2 * D:, :], preferred_element_type=jnp.float32)
    fc = jnp.dot(EC, wvC, preferred_element_type=jnp.float32)
    fl = jnp.dot(EL, wvL, preferred_element_type=jnp.float32)
    fcfl = jnp.concatenate([fc, fl], axis=1)
    w2 = jnp.dot(ME2, fcfl, preferred_element_type=jnp.float32)
    w3 = w2.reshape(N, N, 2)
    uC = jnp.dot(Ht, aC[:D, :], preferred_element_type=jnp.float32)
    vC = jnp.dot(Ht, aC[D:2 * D, :], preferred_element_type=jnp.float32)
    uL = jnp.dot(Ht, aL[:D, :], preferred_element_type=jnp.float32)
    vL = jnp.dot(Ht, aL[D:2 * D, :], preferred_element_type=jnp.float32)
    cstC = jnp.dot(WECb, aC[2 * D:, :], preferred_element_type=jnp.float32)
    cstL = jnp.dot(WELb, aL[2 * D:, :], preferred_element_type=jnp.float32)

    adjN = AH_ref[0] > 0
    scC = _lrelu(uC + jnp.broadcast_to(vC[:, 0], (N, N)) + w3[:, :, 0] + cstC)
    scL = _lrelu(uL + jnp.broadcast_to(vL[:, 0], (N, N)) + w3[:, :, 1] + cstL)
    attnC = _softmax_rows(scC, adjN)
    attnL = _softmax_rows(scL, adjN)
    attnC_sc[...] = attnC
    attnL_sc[...] = attnL

    aggC = jnp.dot(attnC, Ht, preferred_element_type=jnp.float32)
    aggL = jnp.dot(attnL, Ht, preferred_element_type=jnp.float32)

    iota = jax.lax.broadcasted_iota(jnp.int32, (N, 1), 0)
    pm = jnp.zeros((N, 1), dtype=jnp.bool_)
    for k in range(8):
        pm = pm | (iota == pni_ref[k])
    any_adj = jnp.max(AH_ref[0], axis=1, keepdims=True) > 0
    rm = pm & any_adj
    Hn_ref[0] = jnp.where(rm, 0.5 * (aggC + aggL), Ht)

    Hm_ref[0] = jnp.zeros((N, D), dtype=jnp.float32)
    for k in range(8):
        i = pni_ref[k]
        rME = ME_ref[0, pl.ds(i * N, N), :]
        ECr = jnp.dot(rME, FC, preferred_element_type=jnp.float32) + WECb
        ELr = jnp.dot(rME, FL, preferred_element_type=jnp.float32) + WELb
        XC = ECr * Ht
        XL = ELr * Ht
        aCr = attnC_sc[pl.ds(i, 1), :]
        aLr = attnL_sc[pl.ds(i, 1), :]
        mk = 0.5 * (jnp.dot(aCr, XC, preferred_element_type=jnp.float32)
                    + jnp.dot(aLr, XL, preferred_element_type=jnp.float32))
        Hm_ref[0, pl.ds(i, 1), :] = mk
    Hm_ref[0] = Hm_ref[0] * rm.astype(jnp.float32)

    ECt = FC + WECb
    ELt = FL + WELb
    xC = jnp.dot(ECt, bC[:D, :], preferred_element_type=jnp.float32)
    yC = jnp.dot(ECt, bC[D:2 * D, :], preferred_element_type=jnp.float32)
    xL = jnp.dot(ELt, bL[:D, :], preferred_element_type=jnp.float32)
    yL = jnp.dot(ELt, bL[D:2 * D, :], preferred_element_type=jnp.float32)
    zC = jnp.dot(H[0:1, :], bC[2 * D:, :], preferred_element_type=jnp.float32)
    zL = jnp.dot(H[0:1, :], bL[2 * D:, :], preferred_element_type=jnp.float32)

    adjE = AE_ref[0] > 0
    sEC = _lrelu(xC + jnp.broadcast_to(yC[:, 0], (M, M)) + zC)
    sEL = _lrelu(xL + jnp.broadcast_to(yL[:, 0], (M, M)) + zL)
    attnEC = _softmax_rows(sEC, adjE)
    attnEL = _softmax_rows(sEL, adjE)
    aggEC = jnp.dot(attnEC, ECt, preferred_element_type=jnp.float32)
    aggEL = jnp.dot(attnEL, ELt, preferred_element_type=jnp.float32)

    iotaE = jax.lax.broadcasted_iota(jnp.int32, (M, 1), 0)
    pmE = jnp.zeros((M, 1), dtype=jnp.bool_)
    for k in range(8):
        pmE = pmE | (iotaE == pei_ref[k])
    any_adjE = jnp.max(AE_ref[0], axis=1, keepdims=True) > 0
    rmE = pmE & any_adjE
    ECn_ref[0] = jnp.where(rmE, aggEC, ECt)
    ELn_ref[0] = jnp.where(rmE, aggEL, ELt)


def kernel(H, E_C, E_L, AH, AE, ME, MH, path_node_indices, path_edge_indices,
           W_H_w, W_H_b, W_EC_w, W_EC_b, W_EL_w, W_EL_b, a_C, a_L, b_C, b_L):
    B, N, ND = H.shape
    M = E_C.shape[1]
    Fe = E_C.shape[2]
    D = W_H_w.shape[1]
    A3 = a_C.shape[0]
    f32 = jnp.float32

    grid_spec = pltpu.PrefetchScalarGridSpec(
        num_scalar_prefetch=2,
        grid=(B,),
        in_specs=[
            pl.BlockSpec((1, N, ND), lambda b, pni, pei: (b, 0, 0)),
            pl.BlockSpec((1, M, Fe), lambda b, pni, pei: (b, 0, 0)),
            pl.BlockSpec((1, M, Fe), lambda b, pni, pei: (b, 0, 0)),
            pl.BlockSpec((1, N, N), lambda b, pni, pei: (0, 0, 0)),
            pl.BlockSpec((1, M, M), lambda b, pni, pei: (0, 0, 0)),
            pl.BlockSpec((1, N * N, M), lambda b, pni, pei: (b, 0, 0)),
            pl.BlockSpec((ND, D), lambda b, pni, pei: (0, 0)),
            pl.BlockSpec((D,), lambda b, pni, pei: (0,)),
            pl.BlockSpec((Fe, D), lambda b, pni, pei: (0, 0)),
            pl.BlockSpec((D,), lambda b, pni, pei: (0,)),
            pl.BlockSpec((Fe, D), lambda b, pni, pei: (0, 0)),
            pl.BlockSpec((D,), lambda b, pni, pei: (0,)),
            pl.BlockSpec((A3, 1), lambda b, pni, pei: (0, 0)),
            pl.BlockSpec((A3, 1), lambda b, pni, pei: (0, 0)),
            pl.BlockSpec((A3, 1), lambda b, pni, pei: (0, 0)),
            pl.BlockSpec((A3, 1), lambda b, pni, pei: (0, 0)),
        ],
        out_specs=[
            pl.BlockSpec((1, N, D), lambda b, pni, pei: (b, 0, 0)),
            pl.BlockSpec((1, M, D), lambda b, pni, pei: (b, 0, 0)),
            pl.BlockSpec((1, M, D), lambda b, pni, pei: (b, 0, 0)),
            pl.BlockSpec((1, N, D), lambda b, pni, pei: (b, 0, 0)),
        ],
        scratch_shapes=[
            pltpu.VMEM((N, N), f32),
            pltpu.VMEM((N, N), f32),
        ],
    )

    out_shape = [
        jax.ShapeDtypeStruct((B, N, D), f32),
        jax.ShapeDtypeStruct((B, M, D), f32),
        jax.ShapeDtypeStruct((B, M, D), f32),
        jax.ShapeDtypeStruct((B, N, D), f32),
    ]

    Hn, ECn, ELn, Hm = pl.pallas_call(
        _egat_kernel,
        grid_spec=grid_spec,
        out_shape=out_shape,
        compiler_params=pltpu.CompilerParams(
            dimension_semantics=("arbitrary",),
        ),
    )(path_node_indices, path_edge_indices,
      H, E_C, E_L, AH, AE, ME,
      W_H_w, W_H_b, W_EC_w, W_EC_b, W_EL_w, W_EL_b, a_C, a_L, b_C, b_L)

    return (Hn, ECn, ELn, Hm)
